# Initial kernel scaffold; baseline (speedup 1.0000x reference)
import jax
import jax.numpy as jnp
from jax.experimental import pallas as pl


def kernel(obs_pos, same_obs_mask):
    raise NotImplementedError("write your pallas kernel here")


# speedup vs baseline: 1.5658x; 1.5658x over previous
import functools

import jax
import jax.numpy as jnp
from jax import lax
from jax.experimental import pallas as pl
from jax.experimental.pallas import tpu as pltpu
from jax.experimental.pallas import tpu_sc as plsc

_NC = 2
_NS = 16
_NW = _NC * _NS


@functools.lru_cache(maxsize=None)
def _gather_call(n_flat: int):
    e_per_w = n_flat // _NW
    mesh = plsc.VectorSubcoreMesh(core_axis_name="c", subcore_axis_name="s")

    @functools.partial(
        pl.kernel,
        mesh=mesh,
        out_type=jax.ShapeDtypeStruct((n_flat,), jnp.float32),
        scratch_types=[
            pltpu.VMEM((e_per_w,), jnp.int32),
            pltpu.VMEM((e_per_w,), jnp.float32),
            pltpu.SemaphoreType.DMA,
        ],
    )
    def k(table_hbm, idx_hbm, out_hbm, idx_v, vals_v, sem):
        wid = lax.axis_index("s") * _NC + lax.axis_index("c")
        base = wid * e_per_w
        pltpu.sync_copy(idx_hbm.at[pl.ds(base, e_per_w)], idx_v)
        pltpu.async_copy(table_hbm.at[idx_v], vals_v, sem).wait()
        pltpu.sync_copy(vals_v, out_hbm.at[pl.ds(base, e_per_w)])

    return k


@jax.jit
def kernel(obs_pos, same_obs_mask):
    m = same_obs_mask.shape[0]
    d = obs_pos.shape[1]
    idx = same_obs_mask.reshape(-1).astype(jnp.int32)
    idx2 = (idx[:, None] * d + jnp.arange(d, dtype=jnp.int32)).reshape(-1)
    align = 8 * _NW
    n_flat = m * d
    n_pad = ((n_flat + align - 1) // align) * align
    if n_pad != n_flat:
        pad = jnp.arange(n_pad - n_flat, dtype=jnp.int32)
        idx2 = jnp.concatenate([idx2, pad])
    out = _gather_call(n_pad)(obs_pos.reshape(-1).astype(jnp.float32), idx2)
    return out[:n_flat].reshape(m, d)


# speedup vs baseline: 12.5437x; 8.0109x over previous
import functools

import jax
import jax.numpy as jnp
from jax import lax
from jax.experimental import pallas as pl
from jax.experimental.pallas import tpu as pltpu
from jax.experimental.pallas import tpu_sc as plsc

_NC = 2
_NS = 16
_NW = _NC * _NS


@functools.lru_cache(maxsize=None)
def _gather_call(m: int, n: int):
    r = (m // _NW) & ~7
    tail = m - r * _NW
    assert tail % 8 == 0 and tail // 8 <= _NW
    mesh = plsc.VectorSubcoreMesh(core_axis_name="c", subcore_axis_name="s")

    @functools.partial(
        pl.kernel,
        mesh=mesh,
        out_type=(
            jax.ShapeDtypeStruct((m,), jnp.float32),
            jax.ShapeDtypeStruct((m,), jnp.float32),
        ),
        scratch_types=[
            pltpu.VMEM((r,), jnp.int32),
            pltpu.VMEM((r,), jnp.float32),
            pltpu.VMEM((r,), jnp.float32),
            pltpu.VMEM((8,), jnp.int32),
            pltpu.VMEM((8,), jnp.float32),
            pltpu.VMEM((8,), jnp.float32),
            pltpu.SemaphoreType.DMA,
            pltpu.SemaphoreType.DMA,
        ],
    )
    def k(xp_hbm, yp_hbm, idx_hbm, outx_hbm, outy_hbm,
          idx_v, vx, vy, idx_t, vxt, vyt, semx, semy):
        wid = lax.axis_index("s") * _NC + lax.axis_index("c")
        base = wid * r
        pltpu.sync_copy(idx_hbm.at[pl.ds(base, r)], idx_v)
        cx = pltpu.async_copy(xp_hbm.at[idx_v], vx, semx)
        cy = pltpu.async_copy(yp_hbm.at[idx_v], vy, semy)
        cx.wait()
        cy.wait()
        pltpu.sync_copy(vx, outx_hbm.at[pl.ds(base, r)])
        pltpu.sync_copy(vy, outy_hbm.at[pl.ds(base, r)])

        @pl.when(wid < tail // 8)
        def _():
            tbase = r * _NW + wid * 8
            pltpu.sync_copy(idx_hbm.at[pl.ds(tbase, 8)], idx_t)
            tx = pltpu.async_copy(xp_hbm.at[idx_t], vxt, semx)
            ty = pltpu.async_copy(yp_hbm.at[idx_t], vyt, semy)
            tx.wait()
            ty.wait()
            pltpu.sync_copy(vxt, outx_hbm.at[pl.ds(tbase, 8)])
            pltpu.sync_copy(vyt, outy_hbm.at[pl.ds(tbase, 8)])

    return k


@jax.jit
def kernel(obs_pos, same_obs_mask):
    m = same_obs_mask.shape[0]
    n = obs_pos.shape[0]
    idx = same_obs_mask.astype(jnp.int32).reshape(-1)
    xp = obs_pos[:, 0].astype(jnp.float32)
    yp = obs_pos[:, 1].astype(jnp.float32)
    outx, outy = _gather_call(m, n)(xp, yp, idx)
    return jnp.stack([outx, outy], axis=1)


# speedup vs baseline: 15.7103x; 1.2524x over previous
import functools

import jax
import jax.numpy as jnp
from jax import lax
from jax.experimental import pallas as pl
from jax.experimental.pallas import tpu as pltpu
from jax.experimental.pallas import tpu_sc as plsc

_NC = 2
_NS = 16
_SMAX = 12_288


@functools.lru_cache(maxsize=None)
def _gather_call(m: int, n: int):
    r = (m // _NS) & ~15
    tail = m - r * _NS
    assert tail % 8 == 0 and tail // 8 <= _NS
    assert n + 2 * _SMAX + 64 <= 131_000
    sizes = [_SMAX] * (r // _SMAX)
    if r % _SMAX:
        sizes.append(r % _SMAX)
    mesh = plsc.VectorSubcoreMesh(core_axis_name="c", subcore_axis_name="s")

    @functools.partial(
        pl.kernel,
        mesh=mesh,
        compiler_params=pltpu.CompilerParams(needs_layout_passes=False),
        out_type=(
            jax.ShapeDtypeStruct((m,), jnp.float32),
            jax.ShapeDtypeStruct((m,), jnp.float32),
        ),
        scratch_types=[
            pltpu.VMEM((n,), jnp.float32),
            pltpu.VMEM((_SMAX,), jnp.int32),
            pltpu.VMEM((_SMAX,), jnp.float32),
            pltpu.VMEM((16,), jnp.int32),
            pltpu.VMEM((16,), jnp.float32),
            pltpu.SemaphoreType.DMA,
        ],
    )
    def k(xp_hbm, yp_hbm, idx_hbm, outx_hbm, outy_hbm,
          plane_v, idx_v, vals_v, idx_t, vals_t, sem):
        core = lax.axis_index("c")
        sub = lax.axis_index("s")
        base = sub * r
        iota = lax.iota(jnp.int32, 16)

        def run(plane_hbm, out_hbm):
            pltpu.sync_copy(plane_hbm, plane_v)

            off = 0
            for s in sizes:
                pltpu.sync_copy(idx_hbm.at[pl.ds(base + off, s)],
                                idx_v.at[pl.ds(0, s)])

                def body(i, _):
                    pos = idx_v[pl.ds(i * 16, 16)]
                    vals_v[pl.ds(i * 16, 16)] = plsc.load_gather(
                        plane_v, [pos])
                    return 0

                lax.fori_loop(0, s // 16, body, 0, unroll=8)
                pltpu.sync_copy(vals_v.at[pl.ds(0, s)],
                                out_hbm.at[pl.ds(base + off, s)])
                off += s

            if tail:
                @pl.when(sub < tail // 8)
                def _():
                    tb = m - tail + sub * 8
                    pltpu.sync_copy(idx_hbm.at[pl.ds(tb, 8)],
                                    idx_t.at[pl.ds(0, 8)])
                    pos = jnp.where(iota < 8, idx_t[...], 0)
                    vals_t[...] = plsc.load_gather(plane_v, [pos])
                    pltpu.sync_copy(vals_t.at[pl.ds(0, 8)],
                                    out_hbm.at[pl.ds(tb, 8)])

        @pl.when(core == 0)
        def _():
            run(xp_hbm, outx_hbm)

        @pl.when(core == 1)
        def _():
            run(yp_hbm, outy_hbm)

    return k


@jax.jit
def kernel(obs_pos, same_obs_mask):
    m = same_obs_mask.shape[0]
    n = obs_pos.shape[0]
    idx = same_obs_mask.astype(jnp.int32).reshape(-1)
    xp = obs_pos[:, 0].astype(jnp.float32)
    yp = obs_pos[:, 1].astype(jnp.float32)
    outx, outy = _gather_call(m, n)(xp, yp, idx)
    return jnp.stack([outx, outy], axis=1)
